# initial kernel scaffold (unmeasured)
import jax
import jax.numpy as jnp
from jax import lax
from jax.experimental import pallas as pl
from jax.experimental.pallas import tpu as pltpu

_DeviceIdType = getattr(pl, "DeviceIdType", None) or pltpu.DeviceIdType
_sem_signal = getattr(pl, "semaphore_signal", None) or pltpu.semaphore_signal
_sem_wait = getattr(pl, "semaphore_wait", None) or pltpu.semaphore_wait
_CompilerParams = getattr(pltpu, "CompilerParams", None) or pltpu.TPUCompilerParams

M = 4096
D = 2048
HALF = M // 2


def kernel(partial, gamma):
    gamma2d = gamma.reshape(1, D)

    def body(p_ref, g_ref, o_ref, send_buf, recv_buf, send_sem, recv_sem):
        my_x = lax.axis_index("x")
        my_y = lax.axis_index("y")
        my_z = lax.axis_index("z")
        peer = 1 - my_x

        send_buf[...] = p_ref[0, pl.ds(peer * HALF, HALF), :].astype(jnp.bfloat16)

        barrier = pltpu.get_barrier_semaphore()
        _sem_signal(
            barrier,
            inc=1,
            device_id=(peer, my_y, my_z),
            device_id_type=_DeviceIdType.MESH,
        )
        _sem_wait(barrier, 1)

        rdma = pltpu.make_async_remote_copy(
            src_ref=send_buf,
            dst_ref=recv_buf,
            send_sem=send_sem,
            recv_sem=recv_sem,
            device_id=(peer, my_y, my_z),
            device_id_type=_DeviceIdType.MESH,
        )
        rdma.start()
        rdma.wait()

        acc = p_ref[0, pl.ds(my_x * HALF, HALF), :] + recv_buf[...].astype(
            jnp.float32
        )
        rms = jnp.sqrt(jnp.mean(acc * acc, axis=-1, keepdims=True) + 1e-6)
        o_ref[...] = acc / rms * g_ref[...]

    return pl.pallas_call(
        body,
        out_shape=jax.ShapeDtypeStruct((HALF, D), jnp.float32),
        in_specs=[
            pl.BlockSpec(memory_space=pltpu.VMEM),
            pl.BlockSpec(memory_space=pltpu.VMEM),
        ],
        out_specs=pl.BlockSpec(memory_space=pltpu.VMEM),
        scratch_shapes=[
            pltpu.VMEM((HALF, D), jnp.bfloat16),
            pltpu.VMEM((HALF, D), jnp.bfloat16),
            pltpu.SemaphoreType.DMA,
            pltpu.SemaphoreType.DMA,
        ],
        compiler_params=_CompilerParams(collective_id=0),
    )(partial, gamma2d)


# baseline (device time: 122763 ns/iter reference)
import jax
import jax.numpy as jnp
from jax import lax
from jax.experimental import pallas as pl
from jax.experimental.pallas import tpu as pltpu

_DeviceIdType = getattr(pl, "DeviceIdType", None) or pltpu.DeviceIdType
_sem_signal = getattr(pl, "semaphore_signal", None) or pltpu.semaphore_signal
_sem_wait = getattr(pl, "semaphore_wait", None) or pltpu.semaphore_wait
_CompilerParams = getattr(pltpu, "CompilerParams", None) or pltpu.TPUCompilerParams

M = 4096
D = 2048
HALF = M // 2
CH = 512
NCH = HALF // CH


def kernel(partial, gamma):
    gamma2d = gamma.reshape(1, D)

    def body(
        p_ref,
        g_ref,
        o_ref,
        send_buf,
        recv_buf,
        stage,
        copy_sem,
        send_sem,
        recv_sem,
    ):
        my_x = lax.axis_index("x")
        my_y = lax.axis_index("y")
        my_z = lax.axis_index("z")
        peer = 1 - my_x

        for i in range(NCH):
            cp = pltpu.make_async_copy(
                p_ref.at[0, pl.ds(peer * HALF + i * CH, CH), :],
                stage,
                copy_sem,
            )
            cp.start()
            cp.wait()
            send_buf[i * CH : (i + 1) * CH, :] = stage[...].astype(jnp.bfloat16)

        barrier = pltpu.get_barrier_semaphore()
        _sem_signal(
            barrier,
            inc=1,
            device_id=(peer, my_y, my_z),
            device_id_type=_DeviceIdType.MESH,
        )
        _sem_wait(barrier, 1)

        rdma = pltpu.make_async_remote_copy(
            src_ref=send_buf,
            dst_ref=recv_buf,
            send_sem=send_sem,
            recv_sem=recv_sem,
            device_id=(peer, my_y, my_z),
            device_id_type=_DeviceIdType.MESH,
        )
        rdma.start()
        rdma.wait()

        for i in range(NCH):
            cp = pltpu.make_async_copy(
                p_ref.at[0, pl.ds(my_x * HALF + i * CH, CH), :],
                stage,
                copy_sem,
            )
            cp.start()
            cp.wait()
            acc = stage[...] + recv_buf[i * CH : (i + 1) * CH, :].astype(
                jnp.float32
            )
            rms = jnp.sqrt(jnp.mean(acc * acc, axis=-1, keepdims=True) + 1e-6)
            o_ref[i * CH : (i + 1) * CH, :] = acc / rms * g_ref[...]

    return pl.pallas_call(
        body,
        out_shape=jax.ShapeDtypeStruct((HALF, D), jnp.float32),
        in_specs=[
            pl.BlockSpec(memory_space=pl.ANY),
            pl.BlockSpec(memory_space=pltpu.VMEM),
        ],
        out_specs=pl.BlockSpec(memory_space=pltpu.VMEM),
        scratch_shapes=[
            pltpu.VMEM((HALF, D), jnp.bfloat16),
            pltpu.VMEM((HALF, D), jnp.bfloat16),
            pltpu.VMEM((CH, D), jnp.float32),
            pltpu.SemaphoreType.DMA,
            pltpu.SemaphoreType.DMA,
            pltpu.SemaphoreType.DMA,
        ],
        compiler_params=_CompilerParams(collective_id=0),
    )(partial, gamma2d)


# device time: 82050 ns/iter; 1.4962x vs baseline; 1.4962x over previous
import numpy as np

import jax
import jax.numpy as jnp
from jax import lax
from jax.experimental import pallas as pl
from jax.experimental.pallas import tpu as pltpu

_DeviceIdType = getattr(pl, "DeviceIdType", None) or pltpu.DeviceIdType
_sem_signal = getattr(pl, "semaphore_signal", None) or pltpu.semaphore_signal
_sem_wait = getattr(pl, "semaphore_wait", None) or pltpu.semaphore_wait
_CompilerParams = getattr(pltpu, "CompilerParams", None) or pltpu.TPUCompilerParams

M = 4096
D = 2048
HALF = M // 2
NPLANE = 16
BLK = HALF // NPLANE
NCW = 8
NCCW = 7

_RING_ORDER = [
    (0, 0), (0, 1), (0, 2), (0, 3),
    (1, 3), (1, 2), (1, 1),
    (2, 1), (2, 2), (2, 3),
    (3, 3), (3, 2), (3, 1), (3, 0),
    (2, 0), (1, 0),
]
_RPOS = np.zeros((4, 4), np.int32)
for _r, (_y, _z) in enumerate(_RING_ORDER):
    _RPOS[_y, _z] = _r
_RY = np.array([p[0] for p in _RING_ORDER], np.int32)
_RZ = np.array([p[1] for p in _RING_ORDER], np.int32)


def kernel(partial, gamma):
    gamma2d = gamma.reshape(1, D)

    def body(
        p_ref,
        g_ref,
        rpos_ref,
        ry_ref,
        rz_ref,
        o_ref,
        gather,
        stage_mine,
        stage_peer,
        x_send,
        x_recv,
        copy_sem_a,
        copy_sem_b,
        x_send_sem,
        x_recv_sem,
        cw_send_sems,
        cw_recv_sems,
        ccw_send_sems,
        ccw_recv_sems,
    ):
        my_x = lax.axis_index("x")
        my_y = lax.axis_index("y")
        my_z = lax.axis_index("z")
        xpeer = 1 - my_x

        r = rpos_ref[my_y, my_z]
        r_right = lax.rem(r + 1, NPLANE)
        r_left = lax.rem(r + NPLANE - 1, NPLANE)
        right = (my_x, ry_ref[r_right], rz_ref[r_right])
        left = (my_x, ry_ref[r_left], rz_ref[r_left])

        cp_peer = pltpu.make_async_copy(
            p_ref.at[0, pl.ds(xpeer * HALF + r * BLK, BLK), :],
            stage_peer,
            copy_sem_a,
        )
        cp_mine = pltpu.make_async_copy(
            p_ref.at[0, pl.ds(my_x * HALF + r * BLK, BLK), :],
            stage_mine,
            copy_sem_b,
        )
        cp_peer.start()
        cp_mine.start()

        barrier = pltpu.get_barrier_semaphore()
        for nbr in ((xpeer, my_y, my_z), left, right):
            _sem_signal(
                barrier,
                inc=1,
                device_id=nbr,
                device_id_type=_DeviceIdType.MESH,
            )
        _sem_wait(barrier, 3)

        cp_peer.wait()
        x_send[...] = stage_peer[...].astype(jnp.bfloat16)
        xr = pltpu.make_async_remote_copy(
            src_ref=x_send,
            dst_ref=x_recv,
            send_sem=x_send_sem,
            recv_sem=x_recv_sem,
            device_id=(xpeer, my_y, my_z),
            device_id_type=_DeviceIdType.MESH,
        )
        xr.start()
        cp_mine.wait()
        xr.wait()

        acc = stage_mine[...] + x_recv[...].astype(jnp.float32)
        rms = jnp.sqrt(jnp.mean(acc * acc, axis=-1, keepdims=True) + 1e-6)
        own = acc / rms * g_ref[...]
        o_ref[pl.ds(r * BLK, BLK), :] = own
        gather[r, :, :] = own.astype(jnp.bfloat16)

        send_descs = []
        for s in range(NCW):
            cw_blk = lax.rem(r + NPLANE - s, NPLANE)
            cw_send = pltpu.make_async_remote_copy(
                src_ref=gather.at[cw_blk],
                dst_ref=gather.at[cw_blk],
                send_sem=cw_send_sems.at[s],
                recv_sem=cw_recv_sems.at[s],
                device_id=right,
                device_id_type=_DeviceIdType.MESH,
            )
            cw_send.start()
            send_descs.append(cw_send)

            if s < NCCW:
                ccw_blk = lax.rem(r + s, NPLANE)
                ccw_send = pltpu.make_async_remote_copy(
                    src_ref=gather.at[ccw_blk],
                    dst_ref=gather.at[ccw_blk],
                    send_sem=ccw_send_sems.at[s],
                    recv_sem=ccw_recv_sems.at[s],
                    device_id=left,
                    device_id_type=_DeviceIdType.MESH,
                )
                ccw_send.start()
                send_descs.append(ccw_send)

            if s > 0:
                o_ref[pl.ds(cw_blk * BLK, BLK), :] = gather[
                    cw_blk, :, :
                ].astype(jnp.float32)
                prev_ccw = lax.rem(r + s, NPLANE)
                o_ref[pl.ds(prev_ccw * BLK, BLK), :] = gather[
                    prev_ccw, :, :
                ].astype(jnp.float32)

            cw_rblk = lax.rem(r + NPLANE - s - 1, NPLANE)
            cw_recv = pltpu.make_async_remote_copy(
                src_ref=gather.at[cw_rblk],
                dst_ref=gather.at[cw_rblk],
                send_sem=cw_send_sems.at[s],
                recv_sem=cw_recv_sems.at[s],
                device_id=left,
                device_id_type=_DeviceIdType.MESH,
            )
            cw_recv.wait_recv()
            if s < NCCW:
                ccw_rblk = lax.rem(r + s + 1, NPLANE)
                ccw_recv = pltpu.make_async_remote_copy(
                    src_ref=gather.at[ccw_rblk],
                    dst_ref=gather.at[ccw_rblk],
                    send_sem=ccw_send_sems.at[s],
                    recv_sem=ccw_recv_sems.at[s],
                    device_id=right,
                    device_id_type=_DeviceIdType.MESH,
                )
                ccw_recv.wait_recv()

        last_cw = lax.rem(r + NPLANE - NCW, NPLANE)
        o_ref[pl.ds(last_cw * BLK, BLK), :] = gather[last_cw, :, :].astype(
            jnp.float32
        )
        last_ccw = lax.rem(r + NCCW, NPLANE)
        o_ref[pl.ds(last_ccw * BLK, BLK), :] = gather[last_ccw, :, :].astype(
            jnp.float32
        )

        for desc in send_descs:
            desc.wait_send()

    return pl.pallas_call(
        body,
        out_shape=jax.ShapeDtypeStruct((HALF, D), jnp.float32),
        in_specs=[
            pl.BlockSpec(memory_space=pl.ANY),
            pl.BlockSpec(memory_space=pltpu.VMEM),
            pl.BlockSpec(memory_space=pltpu.SMEM),
            pl.BlockSpec(memory_space=pltpu.SMEM),
            pl.BlockSpec(memory_space=pltpu.SMEM),
        ],
        out_specs=pl.BlockSpec(memory_space=pltpu.VMEM),
        scratch_shapes=[
            pltpu.VMEM((NPLANE, BLK, D), jnp.bfloat16),
            pltpu.VMEM((BLK, D), jnp.float32),
            pltpu.VMEM((BLK, D), jnp.float32),
            pltpu.VMEM((BLK, D), jnp.bfloat16),
            pltpu.VMEM((BLK, D), jnp.bfloat16),
            pltpu.SemaphoreType.DMA,
            pltpu.SemaphoreType.DMA,
            pltpu.SemaphoreType.DMA,
            pltpu.SemaphoreType.DMA,
            pltpu.SemaphoreType.DMA((NCW,)),
            pltpu.SemaphoreType.DMA((NCW,)),
            pltpu.SemaphoreType.DMA((NCCW,)),
            pltpu.SemaphoreType.DMA((NCCW,)),
        ],
        compiler_params=_CompilerParams(collective_id=0),
    )(partial, gamma2d, jnp.asarray(_RPOS), jnp.asarray(_RY), jnp.asarray(_RZ))


# device time: 68072 ns/iter; 1.8034x vs baseline; 1.2053x over previous
import numpy as np

import jax
import jax.numpy as jnp
from jax import lax
from jax.experimental import pallas as pl
from jax.experimental.pallas import tpu as pltpu

_DeviceIdType = getattr(pl, "DeviceIdType", None) or pltpu.DeviceIdType
_sem_signal = getattr(pl, "semaphore_signal", None) or pltpu.semaphore_signal
_sem_wait = getattr(pl, "semaphore_wait", None) or pltpu.semaphore_wait
_CompilerParams = getattr(pltpu, "CompilerParams", None) or pltpu.TPUCompilerParams

M = 4096
D = 2048
HALF = M // 2
NPLANE = 16
BLK = HALF // NPLANE
NCW = 8
NCCW = 7
NPC = 2
HBLK = BLK // NPC

_RING_ORDER = [
    (0, 0), (0, 1), (0, 2), (0, 3),
    (1, 3), (1, 2), (1, 1),
    (2, 1), (2, 2), (2, 3),
    (3, 3), (3, 2), (3, 1), (3, 0),
    (2, 0), (1, 0),
]
_RPOS = np.zeros((4, 4), np.int32)
for _r, (_y, _z) in enumerate(_RING_ORDER):
    _RPOS[_y, _z] = _r
_RY = np.array([p[0] for p in _RING_ORDER], np.int32)
_RZ = np.array([p[1] for p in _RING_ORDER], np.int32)


def kernel(partial, gamma):
    gamma2d = gamma.reshape(1, D)

    def body(
        p_ref,
        g_ref,
        rpos_ref,
        ry_ref,
        rz_ref,
        o_ref,
        gather,
        stage_mine,
        stage_peer,
        x_send,
        x_recv,
        copy_sem_a,
        copy_sem_b,
        x_send_sem,
        x_recv_sem,
        cw_send_sems,
        cw_recv_sems,
        ccw_send_sems,
        ccw_recv_sems,
    ):
        my_x = lax.axis_index("x")
        my_y = lax.axis_index("y")
        my_z = lax.axis_index("z")
        xpeer = 1 - my_x

        r = rpos_ref[my_y, my_z]
        r_right = lax.rem(r + 1, NPLANE)
        r_left = lax.rem(r + NPLANE - 1, NPLANE)
        right = (my_x, ry_ref[r_right], rz_ref[r_right])
        left = (my_x, ry_ref[r_left], rz_ref[r_left])

        cp_peer = pltpu.make_async_copy(
            p_ref.at[0, pl.ds(xpeer * HALF + r * BLK, BLK), :],
            stage_peer,
            copy_sem_a,
        )
        cp_mine = pltpu.make_async_copy(
            p_ref.at[0, pl.ds(my_x * HALF + r * BLK, BLK), :],
            stage_mine,
            copy_sem_b,
        )
        cp_peer.start()
        cp_mine.start()

        barrier = pltpu.get_barrier_semaphore()
        for nbr in ((xpeer, my_y, my_z), left, right):
            _sem_signal(
                barrier,
                inc=1,
                device_id=nbr,
                device_id_type=_DeviceIdType.MESH,
            )
        _sem_wait(barrier, 3)

        cp_peer.wait()
        x_send[...] = stage_peer[...].astype(jnp.bfloat16)
        xr = pltpu.make_async_remote_copy(
            src_ref=x_send,
            dst_ref=x_recv,
            send_sem=x_send_sem,
            recv_sem=x_recv_sem,
            device_id=(xpeer, my_y, my_z),
            device_id_type=_DeviceIdType.MESH,
        )
        xr.start()
        cp_mine.wait()
        xr.wait()

        acc = stage_mine[...] + x_recv[...].astype(jnp.float32)
        rms = jnp.sqrt(jnp.mean(acc * acc, axis=-1, keepdims=True) + 1e-6)
        own = acc / rms * g_ref[...]
        o_ref[pl.ds(r * BLK, BLK), :] = own
        gather[r, :, :] = own.astype(jnp.bfloat16)

        send_descs = []

        def _piece(sems_send, sems_recv, t, blk, j, dev):
            return pltpu.make_async_remote_copy(
                src_ref=gather.at[blk, pl.ds(j * HBLK, HBLK), :],
                dst_ref=gather.at[blk, pl.ds(j * HBLK, HBLK), :],
                send_sem=sems_send.at[t],
                recv_sem=sems_recv.at[t],
                device_id=dev,
                device_id_type=_DeviceIdType.MESH,
            )

        def _recv_piece(sems_send, sems_recv, t, blk, j, dev):
            return _piece(sems_send, sems_recv, t, blk, j, dev)

        for s in range(NCW):
            cw_blk = lax.rem(r + NPLANE - s, NPLANE)
            for j in range(NPC):
                t = NPC * s + j
                if t >= NPC:
                    _recv_piece(
                        cw_send_sems, cw_recv_sems, t - NPC, cw_blk, j, left
                    ).wait_recv()
                d = _piece(cw_send_sems, cw_recv_sems, t, cw_blk, j, right)
                d.start()
                send_descs.append(d)

            if s < NCCW:
                ccw_blk = lax.rem(r + s, NPLANE)
                for j in range(NPC):
                    t = NPC * s + j
                    if t >= NPC:
                        _recv_piece(
                            ccw_send_sems, ccw_recv_sems, t - NPC, ccw_blk, j,
                            right,
                        ).wait_recv()
                    d = _piece(
                        ccw_send_sems, ccw_recv_sems, t, ccw_blk, j, left
                    )
                    d.start()
                    send_descs.append(d)

            if s > 0:
                o_ref[pl.ds(cw_blk * BLK, BLK), :] = gather[
                    cw_blk, :, :
                ].astype(jnp.float32)
                if s < NCCW:
                    prev_ccw = lax.rem(r + s, NPLANE)
                    o_ref[pl.ds(prev_ccw * BLK, BLK), :] = gather[
                        prev_ccw, :, :
                    ].astype(jnp.float32)

        last_cw = lax.rem(r + NPLANE - NCW, NPLANE)
        for j in range(NPC):
            _recv_piece(
                cw_send_sems, cw_recv_sems, NPC * (NCW - 1) + j, last_cw, j,
                left,
            ).wait_recv()
        o_ref[pl.ds(last_cw * BLK, BLK), :] = gather[last_cw, :, :].astype(
            jnp.float32
        )
        last_ccw = lax.rem(r + NCCW, NPLANE)
        for j in range(NPC):
            _recv_piece(
                ccw_send_sems, ccw_recv_sems, NPC * (NCCW - 1) + j, last_ccw,
                j, right,
            ).wait_recv()
        o_ref[pl.ds(last_ccw * BLK, BLK), :] = gather[last_ccw, :, :].astype(
            jnp.float32
        )

        for desc in send_descs:
            desc.wait_send()

    return pl.pallas_call(
        body,
        out_shape=jax.ShapeDtypeStruct((HALF, D), jnp.float32),
        in_specs=[
            pl.BlockSpec(memory_space=pl.ANY),
            pl.BlockSpec(memory_space=pltpu.VMEM),
            pl.BlockSpec(memory_space=pltpu.SMEM),
            pl.BlockSpec(memory_space=pltpu.SMEM),
            pl.BlockSpec(memory_space=pltpu.SMEM),
        ],
        out_specs=pl.BlockSpec(memory_space=pltpu.VMEM),
        scratch_shapes=[
            pltpu.VMEM((NPLANE, BLK, D), jnp.bfloat16),
            pltpu.VMEM((BLK, D), jnp.float32),
            pltpu.VMEM((BLK, D), jnp.float32),
            pltpu.VMEM((BLK, D), jnp.bfloat16),
            pltpu.VMEM((BLK, D), jnp.bfloat16),
            pltpu.SemaphoreType.DMA,
            pltpu.SemaphoreType.DMA,
            pltpu.SemaphoreType.DMA,
            pltpu.SemaphoreType.DMA,
            pltpu.SemaphoreType.DMA((NPC * NCW,)),
            pltpu.SemaphoreType.DMA((NPC * NCW,)),
            pltpu.SemaphoreType.DMA((NPC * NCCW,)),
            pltpu.SemaphoreType.DMA((NPC * NCCW,)),
        ],
        compiler_params=_CompilerParams(collective_id=0),
    )(partial, gamma2d, jnp.asarray(_RPOS), jnp.asarray(_RY), jnp.asarray(_RZ))


# device time: 65465 ns/iter; 1.8752x vs baseline; 1.0398x over previous
import numpy as np

import jax
import jax.numpy as jnp
from jax import lax
from jax.experimental import pallas as pl
from jax.experimental.pallas import tpu as pltpu

_DeviceIdType = getattr(pl, "DeviceIdType", None) or pltpu.DeviceIdType
_sem_signal = getattr(pl, "semaphore_signal", None) or pltpu.semaphore_signal
_sem_wait = getattr(pl, "semaphore_wait", None) or pltpu.semaphore_wait
_CompilerParams = getattr(pltpu, "CompilerParams", None) or pltpu.TPUCompilerParams

M = 4096
D = 2048
HALF = M // 2
NPLANE = 16
BLK = HALF // NPLANE
NCW = 8
NCCW = 7
NPC = 2
HBLK = BLK // NPC

_RING_ORDER = [
    (0, 0), (0, 1), (0, 2), (0, 3),
    (1, 3), (1, 2), (1, 1),
    (2, 1), (2, 2), (2, 3),
    (3, 3), (3, 2), (3, 1), (3, 0),
    (2, 0), (1, 0),
]
_RPOS = np.zeros((4, 4), np.int32)
for _r, (_y, _z) in enumerate(_RING_ORDER):
    _RPOS[_y, _z] = _r
_RY = np.array([p[0] for p in _RING_ORDER], np.int32)
_RZ = np.array([p[1] for p in _RING_ORDER], np.int32)


def kernel(partial, gamma):
    gamma2d = gamma.reshape(1, D)

    def body(
        p_ref,
        g_ref,
        rpos_ref,
        ry_ref,
        rz_ref,
        o_ref,
        stage_mine,
        stage_peer,
        x_send,
        x_recv,
        copy_sem_a,
        copy_sem_b,
        x_send_sem,
        x_recv_sem,
        cw_send_sems,
        cw_recv_sems,
        ccw_send_sems,
        ccw_recv_sems,
    ):
        my_x = lax.axis_index("x")
        my_y = lax.axis_index("y")
        my_z = lax.axis_index("z")
        xpeer = 1 - my_x

        r = rpos_ref[my_y, my_z]
        r_right = lax.rem(r + 1, NPLANE)
        r_left = lax.rem(r + NPLANE - 1, NPLANE)
        right = (my_x, ry_ref[r_right], rz_ref[r_right])
        left = (my_x, ry_ref[r_left], rz_ref[r_left])

        cp_peer = pltpu.make_async_copy(
            p_ref.at[0, pl.ds(xpeer * HALF + r * BLK, BLK), :],
            stage_peer,
            copy_sem_a,
        )
        cp_mine = pltpu.make_async_copy(
            p_ref.at[0, pl.ds(my_x * HALF + r * BLK, BLK), :],
            stage_mine,
            copy_sem_b,
        )
        cp_peer.start()
        cp_mine.start()

        barrier = pltpu.get_barrier_semaphore()
        for nbr in ((xpeer, my_y, my_z), left, right):
            _sem_signal(
                barrier,
                inc=1,
                device_id=nbr,
                device_id_type=_DeviceIdType.MESH,
            )
        _sem_wait(barrier, 3)

        cp_peer.wait()
        x_send[...] = stage_peer[...].astype(jnp.bfloat16)
        xr = pltpu.make_async_remote_copy(
            src_ref=x_send,
            dst_ref=x_recv,
            send_sem=x_send_sem,
            recv_sem=x_recv_sem,
            device_id=(xpeer, my_y, my_z),
            device_id_type=_DeviceIdType.MESH,
        )
        xr.start()
        cp_mine.wait()
        xr.wait()

        acc = stage_mine[...] + x_recv[...].astype(jnp.float32)
        rms = jnp.sqrt(jnp.mean(acc * acc, axis=-1, keepdims=True) + 1e-6)
        own = acc / rms * g_ref[...]
        o_ref[pl.ds(r * BLK, BLK), :] = own.astype(jnp.bfloat16)

        send_descs = []

        def _piece(sems_send, sems_recv, t, blk, j, dev):
            rows = pl.ds(blk * BLK + j * HBLK, HBLK)
            return pltpu.make_async_remote_copy(
                src_ref=o_ref.at[rows, :],
                dst_ref=o_ref.at[rows, :],
                send_sem=sems_send.at[t],
                recv_sem=sems_recv.at[t],
                device_id=dev,
                device_id_type=_DeviceIdType.MESH,
            )

        def _recv_piece(sems_send, sems_recv, t, blk, j, dev):
            return _piece(sems_send, sems_recv, t, blk, j, dev)

        for s in range(NCW):
            cw_blk = lax.rem(r + NPLANE - s, NPLANE)
            for j in range(NPC):
                t = NPC * s + j
                if t >= NPC:
                    _recv_piece(
                        cw_send_sems, cw_recv_sems, t - NPC, cw_blk, j, left
                    ).wait_recv()
                d = _piece(cw_send_sems, cw_recv_sems, t, cw_blk, j, right)
                d.start()
                send_descs.append(d)

            if s < NCCW:
                ccw_blk = lax.rem(r + s, NPLANE)
                for j in range(NPC):
                    t = NPC * s + j
                    if t >= NPC:
                        _recv_piece(
                            ccw_send_sems, ccw_recv_sems, t - NPC, ccw_blk, j,
                            right,
                        ).wait_recv()
                    d = _piece(
                        ccw_send_sems, ccw_recv_sems, t, ccw_blk, j, left
                    )
                    d.start()
                    send_descs.append(d)

        last_cw = lax.rem(r + NPLANE - NCW, NPLANE)
        for j in range(NPC):
            _recv_piece(
                cw_send_sems, cw_recv_sems, NPC * (NCW - 1) + j, last_cw, j,
                left,
            ).wait_recv()
        last_ccw = lax.rem(r + NCCW, NPLANE)
        for j in range(NPC):
            _recv_piece(
                ccw_send_sems, ccw_recv_sems, NPC * (NCCW - 1) + j, last_ccw,
                j, right,
            ).wait_recv()

        for desc in send_descs:
            desc.wait_send()

    return pl.pallas_call(
        body,
        out_shape=jax.ShapeDtypeStruct((HALF, D), jnp.bfloat16),
        in_specs=[
            pl.BlockSpec(memory_space=pl.ANY),
            pl.BlockSpec(memory_space=pltpu.VMEM),
            pl.BlockSpec(memory_space=pltpu.SMEM),
            pl.BlockSpec(memory_space=pltpu.SMEM),
            pl.BlockSpec(memory_space=pltpu.SMEM),
        ],
        out_specs=pl.BlockSpec(memory_space=pltpu.VMEM),
        scratch_shapes=[
            pltpu.VMEM((BLK, D), jnp.float32),
            pltpu.VMEM((BLK, D), jnp.float32),
            pltpu.VMEM((BLK, D), jnp.bfloat16),
            pltpu.VMEM((BLK, D), jnp.bfloat16),
            pltpu.SemaphoreType.DMA,
            pltpu.SemaphoreType.DMA,
            pltpu.SemaphoreType.DMA,
            pltpu.SemaphoreType.DMA,
            pltpu.SemaphoreType.DMA((NPC * NCW,)),
            pltpu.SemaphoreType.DMA((NPC * NCW,)),
            pltpu.SemaphoreType.DMA((NPC * NCCW,)),
            pltpu.SemaphoreType.DMA((NPC * NCCW,)),
        ],
        compiler_params=_CompilerParams(collective_id=0),
    )(partial, gamma2d, jnp.asarray(_RPOS), jnp.asarray(_RY), jnp.asarray(_RZ))
